# baseline (speedup 1.0000x reference)
import functools

import jax
import jax.numpy as jnp
import numpy as np
from jax import lax
from jax.experimental import pallas as pl
from jax.experimental.pallas import tpu as pltpu
from jax.experimental.pallas import tpu_sc as plsc

_MIN_RADIUS = 0.05
_MAX_RADIUS = 0.3
_K = 16
_BM = 256

_DENOM = np.float32(4.0 / 3.0 * np.pi * _MIN_RADIUS ** 3 + 1e-08)


def _dist_block(pts_ref, ctr_ref):
    p = pts_ref[0]
    c = ctr_ref[0]
    dx = c[:, 0:1] - p[0:1, :]
    dy = c[:, 1:2] - p[1:2, :]
    dz = c[:, 2:3] - p[2:3, :]
    return jnp.sqrt(dx * dx + dy * dy + dz * dz)


def _count_body(pts_ref, ctr_ref, cnt_ref):
    n = pts_ref.shape[2]
    p = pts_ref[0]
    c = ctr_ref[0]
    acc = jnp.zeros((_BM, 128), jnp.float32)
    for a in range(n // 128):
        lo, hi = a * 128, (a + 1) * 128
        dx = c[:, 0:1] - p[0:1, lo:hi]
        dy = c[:, 1:2] - p[1:2, lo:hi]
        dz = c[:, 2:3] - p[2:3, lo:hi]
        d = jnp.sqrt(dx * dx + dy * dy + dz * dz)
        acc = acc + (d < _MIN_RADIUS).astype(jnp.float32)
    cnt_ref[0, 0] = jnp.sum(acc, axis=1, keepdims=True)


_SUP = np.float32(3e38)
_BIGI = np.float32(1e9)
_R = 3


def _select_body(pts_ref, ctr_ref, cnt_blk_ref, cnt_full_ref, out_ref, v_ref, v2_ref):
    n = pts_ref.shape[2]
    nc = n // 128
    density_full = cnt_full_ref[...] / _DENOM
    density_max = jnp.max(density_full) + np.float32(1e-8)
    density = cnt_blk_ref[0, 0] / _DENOM
    radii = _MIN_RADIUS + (_MAX_RADIUS - _MIN_RADIUS) * (1.0 - density / density_max)
    b_off = pl.program_id(0) * n

    lane_iota = lax.broadcasted_iota(jnp.int32, (_BM, 128), 1).astype(jnp.float32)
    p = pts_ref[0]
    c = ctr_ref[0]
    m = jnp.full((_BM, 128), _SUP, jnp.float32)
    ji = jnp.full((_BM, 128), _BIGI, jnp.float32)
    for a in range(nc):
        lo, hi = a * 128, (a + 1) * 128
        dx = c[:, 0:1] - p[0:1, lo:hi]
        dy = c[:, 1:2] - p[1:2, lo:hi]
        dz = c[:, 2:3] - p[2:3, lo:hi]
        d = jnp.sqrt(dx * dx + dy * dy + dz * dz)
        ch = jnp.where(d < radii, d, jnp.float32(1e10))
        v_ref[:, lo:hi] = ch
        better = ch < m
        m = jnp.where(better, ch, m)
        ji = jnp.where(better, lane_iota + np.float32(a * 128), ji)

    lane_v = []
    lane_i = []
    src = v_ref
    for r in range(_R):
        lane_v.append(m)
        lane_i.append(ji)
        nm = jnp.full((_BM, 128), _SUP, jnp.float32)
        nji = jnp.full((_BM, 128), _BIGI, jnp.float32)
        last = r == _R - 1
        for a in range(nc):
            lo, hi = a * 128, (a + 1) * 128
            ch = src[:, lo:hi]
            ch = jnp.where(lane_iota + np.float32(a * 128) == ji, _SUP, ch)
            if not last:
                v2_ref[:, lo:hi] = ch
            better = ch < nm
            nm = jnp.where(better, ch, nm)
            if not last:
                nji = jnp.where(better, lane_iota + np.float32(a * 128), nji)
        m = nm
        ji = nji
        src = v2_ref
    m4 = m

    w, wi = lane_v[0], lane_i[0]
    lvl = jnp.zeros((_BM, 128), jnp.float32)
    t = None
    for k in range(_K):
        m = jnp.min(w, axis=1, keepdims=True)
        cand = jnp.where(w == m, wi, _BIGI)
        ji = jnp.min(cand, axis=1, keepdims=True)
        out_ref[0, 0, :, k : k + 1] = ji.astype(jnp.int32) + b_off
        t = m
        wn = jnp.where(lvl == 0.0, lane_v[1], jnp.where(lvl == 1.0, lane_v[2], _SUP))
        win = jnp.where(lvl == 0.0, lane_i[1], jnp.where(lvl == 1.0, lane_i[2], _BIGI))
        hit = (w == m) & (wi == ji)
        w = jnp.where(hit, wn, w)
        wi = jnp.where(hit, win, wi)
        lvl = lvl + hit.astype(jnp.float32)

    suspect = (lvl >= float(_R)) & (m4 <= t)
    any_suspect = jnp.max(suspect.astype(jnp.int32))

    @pl.when(any_suspect == 1)
    def _slow():
        iota = lax.broadcasted_iota(jnp.int32, (_BM, n), 1).astype(jnp.float32)
        for k in range(_K):
            v = v_ref[...]
            m = jnp.min(v, axis=1, keepdims=True)
            cand = jnp.where(v == m, iota, _BIGI)
            ji = jnp.min(cand, axis=1, keepdims=True)
            out_ref[0, 0, :, k : k + 1] = ji.astype(jnp.int32) + b_off
            v_ref[...] = jnp.where(iota == ji, _SUP, v)


def _sc_gather_body(
    per_w, ch, feat_ref, idx_ref, out_ref,
    idx_v0, rows_v0, idx_v1, rows_v1, gsem0, gsem1, osem0, osem1,
):
    nc = lax.axis_size("c")
    wid = lax.axis_index("s") * nc + lax.axis_index("c")
    base = wid * per_w
    nch = per_w // ch
    idx_v = [idx_v0, idx_v1]
    rows_v = [rows_v0, rows_v1]
    gsem = [gsem0, gsem1]
    osem = [osem0, osem1]
    gh = [None, None]
    oh = [None, None]
    pltpu.sync_copy(idx_ref.at[pl.ds(base, ch)], idx_v[0])
    gh[0] = pltpu.async_copy(feat_ref.at[idx_v[0]], rows_v[0], gsem[0])
    for i in range(nch):
        cur = i & 1
        nxt = (i + 1) & 1
        if i + 1 < nch:
            if oh[nxt] is not None:
                oh[nxt].wait()
            off2 = base + (i + 1) * ch
            pltpu.sync_copy(idx_ref.at[pl.ds(off2, ch)], idx_v[nxt])
            gh[nxt] = pltpu.async_copy(feat_ref.at[idx_v[nxt]], rows_v[nxt], gsem[nxt])
        gh[cur].wait()
        oh[cur] = pltpu.async_copy(
            rows_v[cur], out_ref.at[pl.ds(base + i * ch, ch)], osem[cur]
        )
    for h in oh:
        if h is not None:
            h.wait()


def kernel(points, features, center_indices):
    B, N, _ = points.shape
    M = center_indices.shape[1]
    C = features.shape[2]
    MB = M // _BM

    pointsT = points.transpose(0, 2, 1)
    centers = jnp.take_along_axis(
        points, jnp.broadcast_to(center_indices[:, :, None], (B, M, 3)), axis=1
    )

    counts = pl.pallas_call(
        _count_body,
        grid=(B, MB),
        in_specs=[
            pl.BlockSpec((1, 3, N), lambda b, mb: (b, 0, 0)),
            pl.BlockSpec((1, _BM, 3), lambda b, mb: (b, mb, 0)),
        ],
        out_specs=pl.BlockSpec((1, 1, _BM, 1), lambda b, mb: (b, mb, 0, 0)),
        out_shape=jax.ShapeDtypeStruct((B, MB, _BM, 1), jnp.float32),
    )(pointsT, centers)

    knn_idx = pl.pallas_call(
        _select_body,
        grid=(B, MB),
        in_specs=[
            pl.BlockSpec((1, 3, N), lambda b, mb: (b, 0, 0)),
            pl.BlockSpec((1, _BM, 3), lambda b, mb: (b, mb, 0)),
            pl.BlockSpec((1, 1, _BM, 1), lambda b, mb: (b, mb, 0, 0)),
            pl.BlockSpec((B, MB, _BM, 1), lambda b, mb: (0, 0, 0, 0)),
        ],
        out_specs=pl.BlockSpec((1, 1, _BM, _K), lambda b, mb: (b, mb, 0, 0)),
        out_shape=jax.ShapeDtypeStruct((B, MB, _BM, _K), jnp.int32),
        scratch_shapes=[
            pltpu.VMEM((_BM, N), jnp.float32),
            pltpu.VMEM((_BM, N), jnp.float32),
        ],
    )(pointsT, centers, counts, counts)

    tot = B * M * _K
    idx_flat = knn_idx.reshape(tot)
    feat_flat = features.reshape(B * N, C)

    info = plsc.get_sparse_core_info()
    nw = info.num_cores * info.num_subcores
    per_w = tot // nw
    ch = 128
    gather = pl.kernel(
        functools.partial(_sc_gather_body, per_w, ch),
        out_type=jax.ShapeDtypeStruct((tot, C), jnp.float32),
        mesh=plsc.VectorSubcoreMesh(core_axis_name="c", subcore_axis_name="s"),
        compiler_params=pltpu.CompilerParams(use_tc_tiling_on_sc=False),
        scratch_types=[
            pltpu.VMEM((ch,), jnp.int32),
            pltpu.VMEM((ch, C), jnp.float32),
            pltpu.VMEM((ch,), jnp.int32),
            pltpu.VMEM((ch, C), jnp.float32),
            pltpu.SemaphoreType.DMA,
            pltpu.SemaphoreType.DMA,
            pltpu.SemaphoreType.DMA,
            pltpu.SemaphoreType.DMA,
        ],
    )
    out_flat = gather(feat_flat, idx_flat)
    return out_flat.reshape(B, M, _K, C)


# speedup vs baseline: 1.5839x; 1.5839x over previous
import functools

import jax
import jax.numpy as jnp
import numpy as np
from jax import lax
from jax.experimental import pallas as pl
from jax.experimental.pallas import tpu as pltpu
from jax.experimental.pallas import tpu_sc as plsc

_MIN_RADIUS = 0.05
_MAX_RADIUS = 0.3
_K = 16
_BM = 128

_DENOM = np.float32(4.0 / 3.0 * np.pi * _MIN_RADIUS ** 3 + 1e-08)


def _count_body(pts_ref, ctr_ref, cnt_ref):
    n = pts_ref.shape[2]
    p = pts_ref[0]
    c = ctr_ref[0]
    acc = jnp.zeros((_BM, 128), jnp.float32)
    for a in range(n // 128):
        lo, hi = a * 128, (a + 1) * 128
        dx = c[:, 0:1] - p[0:1, lo:hi]
        dy = c[:, 1:2] - p[1:2, lo:hi]
        dz = c[:, 2:3] - p[2:3, lo:hi]
        d = jnp.sqrt(dx * dx + dy * dy + dz * dz)
        acc = acc + (d < _MIN_RADIUS).astype(jnp.float32)
    cnt_ref[0, 0] = jnp.sum(acc, axis=1, keepdims=True)


_SUP = np.float32(3e38)
_BIGI = np.float32(1e9)
_R = 3


def _select_body(pts_ref, ctr_ref, cnt_blk_ref, cnt_full_ref, out_ref, v_ref, v2_ref):
    n = pts_ref.shape[2]
    nc = n // 128
    density_full = cnt_full_ref[...] / _DENOM
    density_max = jnp.max(density_full) + np.float32(1e-8)
    density = cnt_blk_ref[0, 0] / _DENOM
    radii = _MIN_RADIUS + (_MAX_RADIUS - _MIN_RADIUS) * (1.0 - density / density_max)
    b_off = pl.program_id(0) * n

    lane_iota = lax.broadcasted_iota(jnp.int32, (_BM, 128), 1).astype(jnp.float32)
    p = pts_ref[0]
    c = ctr_ref[0]
    m = jnp.full((_BM, 128), _SUP, jnp.float32)
    ji = jnp.full((_BM, 128), _BIGI, jnp.float32)
    for a in range(nc):
        lo, hi = a * 128, (a + 1) * 128
        dx = c[:, 0:1] - p[0:1, lo:hi]
        dy = c[:, 1:2] - p[1:2, lo:hi]
        dz = c[:, 2:3] - p[2:3, lo:hi]
        d = jnp.sqrt(dx * dx + dy * dy + dz * dz)
        ch = jnp.where(d < radii, d, jnp.float32(1e10))
        v_ref[:, lo:hi] = ch
        better = ch < m
        m = jnp.where(better, ch, m)
        ji = jnp.where(better, lane_iota + np.float32(a * 128), ji)

    lane_v = []
    lane_i = []
    src = v_ref
    for r in range(_R):
        lane_v.append(m)
        lane_i.append(ji)
        nm = jnp.full((_BM, 128), _SUP, jnp.float32)
        nji = jnp.full((_BM, 128), _BIGI, jnp.float32)
        last = r == _R - 1
        for a in range(nc):
            lo, hi = a * 128, (a + 1) * 128
            ch = src[:, lo:hi]
            ch = jnp.where(lane_iota + np.float32(a * 128) == ji, _SUP, ch)
            if not last:
                v2_ref[:, lo:hi] = ch
            better = ch < nm
            nm = jnp.where(better, ch, nm)
            if not last:
                nji = jnp.where(better, lane_iota + np.float32(a * 128), nji)
        m = nm
        ji = nji
        src = v2_ref
    m4 = m

    w, wi = lane_v[0], lane_i[0]
    lvl = jnp.zeros((_BM, 128), jnp.float32)
    t = None
    for k in range(_K):
        m = jnp.min(w, axis=1, keepdims=True)
        cand = jnp.where(w == m, wi, _BIGI)
        ji = jnp.min(cand, axis=1, keepdims=True)
        out_ref[0, 0, :, k : k + 1] = ji.astype(jnp.int32) + b_off
        t = m
        wn = jnp.where(lvl == 0.0, lane_v[1], jnp.where(lvl == 1.0, lane_v[2], _SUP))
        win = jnp.where(lvl == 0.0, lane_i[1], jnp.where(lvl == 1.0, lane_i[2], _BIGI))
        hit = (w == m) & (wi == ji)
        w = jnp.where(hit, wn, w)
        wi = jnp.where(hit, win, wi)
        lvl = lvl + hit.astype(jnp.float32)

    suspect = (lvl >= float(_R)) & (m4 <= t)
    any_suspect = jnp.max(suspect.astype(jnp.int32))

    @pl.when(any_suspect == 1)
    def _slow():
        iota = lax.broadcasted_iota(jnp.int32, (_BM, n), 1).astype(jnp.float32)
        for k in range(_K):
            v = v_ref[...]
            m = jnp.min(v, axis=1, keepdims=True)
            cand = jnp.where(v == m, iota, _BIGI)
            ji = jnp.min(cand, axis=1, keepdims=True)
            out_ref[0, 0, :, k : k + 1] = ji.astype(jnp.int32) + b_off
            v_ref[...] = jnp.where(iota == ji, _SUP, v)


def _sc_gather_body(
    per_w, ch, feat_ref, idx_ref, out_ref,
    idx_v0, rows_v0, idx_v1, rows_v1, gsem0, gsem1, osem0, osem1,
):
    nc = lax.axis_size("c")
    wid = lax.axis_index("s") * nc + lax.axis_index("c")
    base = wid * per_w
    nch = per_w // ch
    idx_v = [idx_v0, idx_v1]
    rows_v = [rows_v0, rows_v1]
    gsem = [gsem0, gsem1]
    osem = [osem0, osem1]
    gh = [None, None]
    oh = [None, None]
    pltpu.sync_copy(idx_ref.at[pl.ds(base, ch)], idx_v[0])
    gh[0] = pltpu.async_copy(feat_ref.at[idx_v[0]], rows_v[0], gsem[0])
    for i in range(nch):
        cur = i & 1
        nxt = (i + 1) & 1
        if i + 1 < nch:
            if oh[nxt] is not None:
                oh[nxt].wait()
            off2 = base + (i + 1) * ch
            pltpu.sync_copy(idx_ref.at[pl.ds(off2, ch)], idx_v[nxt])
            gh[nxt] = pltpu.async_copy(feat_ref.at[idx_v[nxt]], rows_v[nxt], gsem[nxt])
        gh[cur].wait()
        oh[cur] = pltpu.async_copy(
            rows_v[cur], out_ref.at[pl.ds(base + i * ch, ch)], osem[cur]
        )
    for h in oh:
        if h is not None:
            h.wait()


def kernel(points, features, center_indices):
    B, N, _ = points.shape
    M = center_indices.shape[1]
    C = features.shape[2]
    MB = M // _BM

    pointsT = points.transpose(0, 2, 1)
    centers = jnp.take_along_axis(
        points, jnp.broadcast_to(center_indices[:, :, None], (B, M, 3)), axis=1
    )

    counts = pl.pallas_call(
        _count_body,
        grid=(B, MB),
        in_specs=[
            pl.BlockSpec((1, 3, N), lambda b, mb: (b, 0, 0)),
            pl.BlockSpec((1, _BM, 3), lambda b, mb: (b, mb, 0)),
        ],
        out_specs=pl.BlockSpec((1, 1, _BM, 1), lambda b, mb: (b, mb, 0, 0)),
        out_shape=jax.ShapeDtypeStruct((B, MB, _BM, 1), jnp.float32),
    )(pointsT, centers)

    knn_idx = pl.pallas_call(
        _select_body,
        grid=(B, MB),
        in_specs=[
            pl.BlockSpec((1, 3, N), lambda b, mb: (b, 0, 0)),
            pl.BlockSpec((1, _BM, 3), lambda b, mb: (b, mb, 0)),
            pl.BlockSpec((1, 1, _BM, 1), lambda b, mb: (b, mb, 0, 0)),
            pl.BlockSpec((B, MB, _BM, 1), lambda b, mb: (0, 0, 0, 0)),
        ],
        out_specs=pl.BlockSpec((1, 1, _BM, _K), lambda b, mb: (b, mb, 0, 0)),
        out_shape=jax.ShapeDtypeStruct((B, MB, _BM, _K), jnp.int32),
        scratch_shapes=[
            pltpu.VMEM((_BM, N), jnp.float32),
            pltpu.VMEM((_BM, N), jnp.float32),
        ],
    )(pointsT, centers, counts, counts)

    tot = B * M * _K
    idx_flat = knn_idx.reshape(tot)
    feat_flat = features.reshape(B * N, C)

    info = plsc.get_sparse_core_info()
    nw = info.num_cores * info.num_subcores
    per_w = tot // nw
    ch = 128
    gather = pl.kernel(
        functools.partial(_sc_gather_body, per_w, ch),
        out_type=jax.ShapeDtypeStruct((tot, C), jnp.float32),
        mesh=plsc.VectorSubcoreMesh(core_axis_name="c", subcore_axis_name="s"),
        compiler_params=pltpu.CompilerParams(use_tc_tiling_on_sc=False),
        scratch_types=[
            pltpu.VMEM((ch,), jnp.int32),
            pltpu.VMEM((ch, C), jnp.float32),
            pltpu.VMEM((ch,), jnp.int32),
            pltpu.VMEM((ch, C), jnp.float32),
            pltpu.SemaphoreType.DMA,
            pltpu.SemaphoreType.DMA,
            pltpu.SemaphoreType.DMA,
            pltpu.SemaphoreType.DMA,
        ],
    )
    out_flat = gather(feat_flat, idx_flat)
    return out_flat.reshape(B, M, _K, C)
